# baseline (speedup 1.0000x reference)
import numpy as np

import jax
import jax.numpy as jnp
from jax.experimental import pallas as pl
from jax.experimental.pallas import tpu as pltpu

N = 10000
D = 256
T = 1024
NT = 10
NP = NT * T
PAD = NP - N

_R, _C, _DE1, _DE2, _WE1, _WE2, _ZP, _STG, _O2I = range(9)


def _make_schedule(nt: int) -> np.ndarray:
    steps = []
    for k in range(nt):
        steps.append((k, k, 1, 0, 0, 0, 0, k, 0))
        for j in range(k + 1, nt):
            steps.append((k, j, 1, 0, 0, 0, 0, k, 0))
        last = 1 if k == nt - 1 else 0
        for i in range(nt):
            steps.append((i, k, 1 if i > k else 0, 1, 1 if i == 0 else 0,
                          last, 1 if (last and i == 0) else 0, k,
                          i if last else 0))
    return np.asarray(steps, dtype=np.int32)


_SCHEDULE = _make_schedule(NT)


def _tri_kernel(sref, a_ref, x0_ref, e1o_ref, e2o_ref, os_ref, e1_s, e2_s):
    t = pl.program_id(0)
    r = sref[t, _R]
    c = sref[t, _C]

    @pl.when(t == 0)
    def _():
        e1_s[...] = jnp.zeros_like(e1_s)
        e2_s[...] = jnp.zeros_like(e2_s)

    if PAD:
        @pl.when(sref[t, _ZP] == 1)
        def _():
            e1_s[pl.ds(N, PAD), :] = jnp.zeros((PAD, D), jnp.float32)

    a = a_ref[...]

    @pl.when(sref[t, _DE1] == 1)
    def _():
        e1_s[pl.ds(r * T, T), :] += jnp.dot(
            a, x0_ref[pl.ds(c * T, T), :], preferred_element_type=jnp.float32)

    @pl.when(sref[t, _DE2] == 1)
    def _():
        e2_s[pl.ds(r * T, T), :] += jnp.dot(
            a, e1_s[pl.ds(c * T, T), :], preferred_element_type=jnp.float32)

    @pl.when(sref[t, _WE1] == 1)
    def _():
        k = sref[t, _STG]
        e1o_ref[...] = e1_s[pl.ds(k * T, T), :]

    @pl.when(sref[t, _WE2] == 1)
    def _():
        e2_blk = e2_s[pl.ds(r * T, T), :]
        e2o_ref[...] = e2_blk
        os_ref[...] = (x0_ref[pl.ds(r * T, T), :]
                       + e1_s[pl.ds(r * T, T), :] + e2_blk)


def kernel(encoder_adj, init_emb):
    x0p = jnp.pad(init_emb, ((0, PAD), (0, 0)))
    sched = jnp.asarray(_SCHEDULE)

    a_spec = pl.BlockSpec((T, T), lambda t, s: (s[t, _R], s[t, _C]))
    x0_spec = pl.BlockSpec((NP, D), lambda t, s: (0, 0))
    e1o_spec = pl.BlockSpec((T, D), lambda t, s: (s[t, _STG], 0))
    out2_spec = pl.BlockSpec((T, D), lambda t, s: (s[t, _O2I], 0))

    grid_spec = pltpu.PrefetchScalarGridSpec(
        num_scalar_prefetch=1,
        grid=(_SCHEDULE.shape[0],),
        in_specs=[a_spec, x0_spec],
        out_specs=[e1o_spec, out2_spec, out2_spec],
        scratch_shapes=[pltpu.VMEM((NP, D), jnp.float32),
                        pltpu.VMEM((NP, D), jnp.float32)],
    )

    e1, e2, summed = pl.pallas_call(
        _tri_kernel,
        grid_spec=grid_spec,
        out_shape=[
            jax.ShapeDtypeStruct((N, D), jnp.float32),
            jax.ShapeDtypeStruct((N, D), jnp.float32),
            jax.ShapeDtypeStruct((N, D), jnp.float32),
        ],
    )(sched, encoder_adj, x0p)

    return (summed, init_emb, e1, e2)


# speedup vs baseline: 1.2105x; 1.2105x over previous
import jax
import jax.numpy as jnp
from jax.experimental import pallas as pl
from jax.experimental.pallas import tpu as pltpu

N = 10000
D = 256
BM = 400
NB = N // BM


def _fused_kernel(a_ref, x0_full_ref, e1_ref, e2_ref,
                  osum_ref, e1_scratch):
    i = pl.program_id(0)

    @pl.when(i < NB)
    def _():
        e1_blk = jnp.dot(a_ref[...], x0_full_ref[...],
                         preferred_element_type=jnp.float32)
        e1_ref[...] = e1_blk
        e1_scratch[pl.ds(i * BM, BM), :] = e1_blk

    @pl.when(i >= NB)
    def _():
        j = i - NB
        e2_blk = jnp.dot(a_ref[...], e1_scratch[...],
                         preferred_element_type=jnp.float32)
        e2_ref[...] = e2_blk
        osum_ref[...] = (
            x0_full_ref[pl.ds(j * BM, BM), :]
            + e1_scratch[pl.ds(j * BM, BM), :] + e2_blk)


def kernel(encoder_adj, init_emb):
    a_spec = pl.BlockSpec((BM, N), lambda i: (i % NB, 0))
    x0_full_spec = pl.BlockSpec((N, D), lambda i: (0, 0))
    e1_spec = pl.BlockSpec((BM, D), lambda i: (jnp.minimum(i, NB - 1), 0))
    out2_spec = pl.BlockSpec((BM, D), lambda i: (jnp.maximum(i - NB, 0), 0))

    e1, e2, summed = pl.pallas_call(
        _fused_kernel,
        grid=(2 * NB,),
        in_specs=[a_spec, x0_full_spec],
        out_specs=[e1_spec, out2_spec, out2_spec],
        out_shape=[
            jax.ShapeDtypeStruct((N, D), jnp.float32),
            jax.ShapeDtypeStruct((N, D), jnp.float32),
            jax.ShapeDtypeStruct((N, D), jnp.float32),
        ],
        scratch_shapes=[pltpu.VMEM((N, D), jnp.float32)],
    )(encoder_adj, init_emb)

    return (summed, init_emb, e1, e2)
